# baseline (speedup 1.0000x reference)
import functools

import jax
import jax.numpy as jnp
from jax import lax
from jax.experimental import pallas as pl
from jax.experimental.pallas import tpu as pltpu
from jax.experimental.pallas import tpu_sc as plsc

N_NODES = 10000
N_PAD = 10240
D_IN = 128
E_EDGES = 320000
CH = 128
NCHUNK = 79
E_PAD = 2 * 16 * NCHUNK * CH
DEG_W = 128
ROWS_PER_TILE = N_PAD // 16

_MESH = plsc.VectorSubcoreMesh(core_axis_name="c", subcore_axis_name="s")


@functools.partial(
    pl.kernel,
    mesh=_MESH,
    out_type=jax.ShapeDtypeStruct((2, N_PAD, DEG_W), jnp.float32),
    scratch_types=[
        pltpu.VMEM((NCHUNK, CH), jnp.int32),
        pltpu.VMEM((CH, DEG_W), jnp.float32),
        pltpu.VMEM((CH, DEG_W), jnp.float32),
        pltpu.VMEM_SHARED((N_PAD, DEG_W), jnp.float32),
    ],
)
def _sc_degree(col_hbm, out_hbm, cidx, ones_v, zeros_v, dacc):
    c = lax.axis_index("c")
    s = lax.axis_index("s")
    one16 = jnp.ones((16,), jnp.float32)
    zero16 = jnp.zeros((16,), jnp.float32)

    def _fill(i, _):
        for j in range(DEG_W // 16):
            ones_v[i, pl.ds(j * 16, 16)] = one16
            zeros_v[i, pl.ds(j * 16, 16)] = zero16
        return 0

    lax.fori_loop(0, CH, _fill, 0)
    for blk in range(ROWS_PER_TILE // CH):
        pltpu.sync_copy(zeros_v, dacc.at[pl.ds(s * ROWS_PER_TILE + blk * CH, CH)])
    pltpu.sync_copy(col_hbm.at[c, s], cidx)
    plsc.subcore_barrier()

    def _body(j, _):
        pltpu.sync_copy(ones_v, dacc.at[cidx.at[j]], add=True)
        return 0

    lax.fori_loop(0, NCHUNK, _body, 0)
    plsc.subcore_barrier()
    pltpu.sync_copy(dacc.at[pl.ds(s * ROWS_PER_TILE, ROWS_PER_TILE)],
                    out_hbm.at[c, pl.ds(s * ROWS_PER_TILE, ROWS_PER_TILE)])


@functools.partial(
    pl.kernel,
    mesh=_MESH,
    out_type=jax.ShapeDtypeStruct((2, N_PAD, D_IN), jnp.float32),
    scratch_types=[
        pltpu.VMEM((NCHUNK, CH), jnp.int32),
        pltpu.VMEM((NCHUNK, CH), jnp.int32),
        pltpu.VMEM((CH, D_IN), jnp.float32),
        pltpu.VMEM_SHARED((N_PAD, D_IN), jnp.float32),
        pltpu.SemaphoreType.DMA,
    ],
)
def _sc_scatter(vals_hbm, row_hbm, col_hbm, out_hbm, ridx, cidx, rows0,
                acc, g0):
    c = lax.axis_index("c")
    s = lax.axis_index("s")
    zero16 = jnp.zeros((16,), jnp.float32)

    def _zrow(i, _):
        for j in range(D_IN // 16):
            rows0[i, pl.ds(j * 16, 16)] = zero16
        return 0

    lax.fori_loop(0, CH, _zrow, 0)
    for blk in range(ROWS_PER_TILE // CH):
        pltpu.sync_copy(rows0, acc.at[pl.ds(s * ROWS_PER_TILE + blk * CH, CH)])
    pltpu.sync_copy(row_hbm.at[c, s], ridx)
    pltpu.sync_copy(col_hbm.at[c, s], cidx)
    plsc.subcore_barrier()

    def _body(j, _):
        pltpu.async_copy(vals_hbm.at[ridx.at[j]], rows0, g0).wait()
        pltpu.sync_copy(rows0, acc.at[cidx.at[j]], add=True)
        return 0

    lax.fori_loop(0, NCHUNK, _body, 0)
    plsc.subcore_barrier()
    pltpu.sync_copy(acc.at[pl.ds(s * ROWS_PER_TILE, ROWS_PER_TILE)],
                    out_hbm.at[c, pl.ds(s * ROWS_PER_TILE, ROWS_PER_TILE)])


_RB = 1280


def _dis_block(d0, d1):
    deg = d0[:, 0:1] + d1[:, 0:1] + 1.0
    return lax.rsqrt(deg)


def _prescale_body(d0, d1, x_ref, o_ref):
    o_ref[...] = x_ref[...] * _dis_block(d0, d1)


def _mid_body(d0, d1, a0, a1, xp_ref, w1, b1, w2, o_ref):
    dis = _dis_block(d0, d1)
    s1 = (a0[...] + a1[...] + xp_ref[...]) * dis
    h = jnp.maximum(
        jnp.dot(s1, w1[...], preferred_element_type=jnp.float32) + b1[...], 0.0)
    g = jnp.dot(h, w2[...], preferred_element_type=jnp.float32)
    o_ref[...] = g * dis


def _final_body(d0, d1, a0, a1, gp_ref, b2, o_ref):
    dis = _dis_block(d0, d1)
    o_ref[...] = (a0[...] + a1[...] + gp_ref[...]) * dis + b2[...]


def _row_spec(width):
    return pl.BlockSpec((_RB, width), lambda i: (i, 0))


def _full_spec(shape):
    return pl.BlockSpec(shape, lambda i: tuple(0 for _ in shape))


def kernel(x, edge_index, conv1_weight, conv1_bias, conv2_weight, conv2_bias):
    ei = edge_index.astype(jnp.int32)
    pad = jnp.full((E_PAD - E_EDGES,), N_NODES, jnp.int32)
    row = jnp.concatenate([ei[0], pad]).reshape(2, 16, NCHUNK, CH)
    col = jnp.concatenate([ei[1], pad]).reshape(2, 16, NCHUNK, CH)
    x_pad = jnp.zeros((N_PAD, D_IN), x.dtype).at[:N_NODES].set(x)
    b1 = conv1_bias.reshape(1, -1)
    b2 = conv2_bias.reshape(1, -1)

    deg2 = _sc_degree(col)
    d0, d1 = deg2[0], deg2[1]

    grid = (N_PAD // _RB,)
    xp = pl.pallas_call(
        _prescale_body,
        grid=grid,
        in_specs=[_row_spec(DEG_W), _row_spec(DEG_W), _row_spec(D_IN)],
        out_specs=_row_spec(D_IN),
        out_shape=jax.ShapeDtypeStruct((N_PAD, D_IN), jnp.float32),
    )(d0, d1, x_pad)

    acc1 = _sc_scatter(xp, row, col)

    gp = pl.pallas_call(
        _mid_body,
        grid=grid,
        in_specs=[
            _row_spec(DEG_W), _row_spec(DEG_W),
            _row_spec(D_IN), _row_spec(D_IN), _row_spec(D_IN),
            _full_spec(conv1_weight.shape), _full_spec(b1.shape),
            _full_spec(conv2_weight.shape),
        ],
        out_specs=_row_spec(D_IN),
        out_shape=jax.ShapeDtypeStruct((N_PAD, D_IN), jnp.float32),
    )(d0, d1, acc1[0], acc1[1], xp, conv1_weight, b1, conv2_weight)

    acc2 = _sc_scatter(gp, row, col)

    out = pl.pallas_call(
        _final_body,
        grid=grid,
        in_specs=[
            _row_spec(DEG_W), _row_spec(DEG_W),
            _row_spec(D_IN), _row_spec(D_IN), _row_spec(D_IN),
            _full_spec(b2.shape),
        ],
        out_specs=_row_spec(D_IN),
        out_shape=jax.ShapeDtypeStruct((N_PAD, D_IN), jnp.float32),
    )(d0, d1, acc2[0], acc2[1], gp, b2)

    return out[:N_NODES]


# speedup vs baseline: 2.1233x; 2.1233x over previous
import functools

import jax
import jax.numpy as jnp
from jax import lax
from jax.experimental import pallas as pl
from jax.experimental.pallas import tpu as pltpu
from jax.experimental.pallas import tpu_sc as plsc

N_NODES = 10000
N_PAD = 10240
D_IN = 128
E_EDGES = 320000
CH = 128
NCHUNK = 80
E_PAD = 2 * 16 * NCHUNK * CH
DEG_W = 128
ROWS_PER_TILE = N_PAD // 16

_MESH = plsc.VectorSubcoreMesh(core_axis_name="c", subcore_axis_name="s")


@functools.partial(
    pl.kernel,
    mesh=_MESH,
    out_type=jax.ShapeDtypeStruct((2, N_PAD, DEG_W), jnp.float32),
    scratch_types=[
        pltpu.VMEM((NCHUNK, CH), jnp.int32),
        pltpu.VMEM((CH, DEG_W), jnp.float32),
        pltpu.VMEM((CH, DEG_W), jnp.float32),
        pltpu.VMEM_SHARED((N_PAD, DEG_W), jnp.float32),
    ],
)
def _sc_degree(col_hbm, out_hbm, cidx, ones_v, zeros_v, dacc):
    c = lax.axis_index("c")
    s = lax.axis_index("s")
    one16 = jnp.ones((16,), jnp.float32)
    zero16 = jnp.zeros((16,), jnp.float32)

    def _fill(i, _):
        for j in range(DEG_W // 16):
            ones_v[i, pl.ds(j * 16, 16)] = one16
            zeros_v[i, pl.ds(j * 16, 16)] = zero16
        return 0

    lax.fori_loop(0, CH, _fill, 0)
    for blk in range(ROWS_PER_TILE // CH):
        pltpu.sync_copy(zeros_v, dacc.at[pl.ds(s * ROWS_PER_TILE + blk * CH, CH)])
    pltpu.sync_copy(col_hbm.at[c, s], cidx)
    plsc.subcore_barrier()

    def _body(j, _):
        pltpu.sync_copy(ones_v, dacc.at[cidx.at[j]], add=True)
        return 0

    lax.fori_loop(0, NCHUNK, _body, 0)
    plsc.subcore_barrier()
    pltpu.sync_copy(dacc.at[pl.ds(s * ROWS_PER_TILE, ROWS_PER_TILE)],
                    out_hbm.at[c, pl.ds(s * ROWS_PER_TILE, ROWS_PER_TILE)])


@functools.partial(
    pl.kernel,
    mesh=_MESH,
    out_type=jax.ShapeDtypeStruct((2, N_PAD, D_IN), jnp.float32),
    scratch_types=[
        pltpu.VMEM((NCHUNK // 2, CH), jnp.int32),
        pltpu.VMEM((NCHUNK // 2, CH), jnp.int32),
        pltpu.VMEM((CH, D_IN), jnp.float32),
        pltpu.VMEM((CH, D_IN), jnp.float32),
        pltpu.VMEM_SHARED((N_PAD, D_IN), jnp.float32),
        pltpu.SemaphoreType.DMA,
        pltpu.SemaphoreType.DMA,
    ],
)
def _sc_scatter(vals_hbm, row_hbm, col_hbm, out_hbm, ridx, cidx, rows0, rows1,
                acc, g0, g1):
    c = lax.axis_index("c")
    s = lax.axis_index("s")
    zero16 = jnp.zeros((16,), jnp.float32)
    half = NCHUNK // 2

    def _zrow(i, _):
        for j in range(D_IN // 16):
            rows0[i, pl.ds(j * 16, 16)] = zero16
        return 0

    lax.fori_loop(0, CH, _zrow, 0)
    for blk in range(ROWS_PER_TILE // CH):
        pltpu.sync_copy(rows0, acc.at[pl.ds(s * ROWS_PER_TILE + blk * CH, CH)])
    plsc.subcore_barrier()

    for h in range(2):
        pltpu.sync_copy(row_hbm.at[c, s, h], ridx)
        pltpu.sync_copy(col_hbm.at[c, s, h], cidx)
        pltpu.async_copy(vals_hbm.at[ridx.at[0]], rows0, g0)
        pltpu.async_copy(vals_hbm.at[ridx.at[1]], rows1, g1)

        def _body(k, _):
            j = 2 * k
            pltpu.make_async_copy(vals_hbm.at[ridx.at[j]], rows0, g0).wait()
            pltpu.sync_copy(rows0, acc.at[cidx.at[j]], add=True)
            pltpu.async_copy(vals_hbm.at[ridx.at[j + 2]], rows0, g0)
            pltpu.make_async_copy(vals_hbm.at[ridx.at[j + 1]], rows1, g1).wait()
            pltpu.sync_copy(rows1, acc.at[cidx.at[j + 1]], add=True)
            pltpu.async_copy(vals_hbm.at[ridx.at[j + 3]], rows1, g1)
            return 0

        lax.fori_loop(0, half // 2 - 1, _body, 0)
        jlast = half - 2
        pltpu.make_async_copy(vals_hbm.at[ridx.at[jlast]], rows0, g0).wait()
        pltpu.sync_copy(rows0, acc.at[cidx.at[jlast]], add=True)
        pltpu.make_async_copy(vals_hbm.at[ridx.at[jlast + 1]], rows1, g1).wait()
        pltpu.sync_copy(rows1, acc.at[cidx.at[jlast + 1]], add=True)
    plsc.subcore_barrier()
    pltpu.sync_copy(acc.at[pl.ds(s * ROWS_PER_TILE, ROWS_PER_TILE)],
                    out_hbm.at[c, pl.ds(s * ROWS_PER_TILE, ROWS_PER_TILE)])


_RB = 1280


def _dis_block(d0, d1):
    deg = d0[:, 0:1] + d1[:, 0:1] + 1.0
    return lax.rsqrt(deg)


def _prescale_body(d0, d1, x_ref, o_ref):
    o_ref[...] = x_ref[...] * _dis_block(d0, d1)


def _mid_body(d0, d1, a0, a1, xp_ref, w1, b1, w2, o_ref):
    dis = _dis_block(d0, d1)
    s1 = (a0[...] + a1[...] + xp_ref[...]) * dis
    h = jnp.maximum(
        jnp.dot(s1, w1[...], preferred_element_type=jnp.float32) + b1[...], 0.0)
    g = jnp.dot(h, w2[...], preferred_element_type=jnp.float32)
    o_ref[...] = g * dis


def _final_body(d0, d1, a0, a1, gp_ref, b2, o_ref):
    dis = _dis_block(d0, d1)
    o_ref[...] = (a0[...] + a1[...] + gp_ref[...]) * dis + b2[...]


def _row_spec(width):
    return pl.BlockSpec((_RB, width), lambda i: (i, 0))


def _full_spec(shape):
    return pl.BlockSpec(shape, lambda i: tuple(0 for _ in shape))


def kernel(x, edge_index, conv1_weight, conv1_bias, conv2_weight, conv2_bias):
    ei = edge_index.astype(jnp.int32)
    pad_i = jnp.arange(E_PAD - E_EDGES, dtype=jnp.int32)
    pad_src = pad_i % N_PAD
    pad_dst = N_NODES + pad_i % (N_PAD - N_NODES)
    row = jnp.concatenate([ei[0], pad_src]).reshape(2, 16, 2, NCHUNK // 2, CH)
    col = jnp.concatenate([ei[1], pad_dst]).reshape(2, 16, 2, NCHUNK // 2, CH)
    col4 = col.reshape(2, 16, NCHUNK, CH)
    x_pad = jnp.zeros((N_PAD, D_IN), x.dtype).at[:N_NODES].set(x)
    b1 = conv1_bias.reshape(1, -1)
    b2 = conv2_bias.reshape(1, -1)

    deg2 = _sc_degree(col4)
    d0, d1 = deg2[0], deg2[1]

    grid = (N_PAD // _RB,)
    xp = pl.pallas_call(
        _prescale_body,
        grid=grid,
        in_specs=[_row_spec(DEG_W), _row_spec(DEG_W), _row_spec(D_IN)],
        out_specs=_row_spec(D_IN),
        out_shape=jax.ShapeDtypeStruct((N_PAD, D_IN), jnp.float32),
    )(d0, d1, x_pad)

    acc1 = _sc_scatter(xp, row, col)

    gp = pl.pallas_call(
        _mid_body,
        grid=grid,
        in_specs=[
            _row_spec(DEG_W), _row_spec(DEG_W),
            _row_spec(D_IN), _row_spec(D_IN), _row_spec(D_IN),
            _full_spec(conv1_weight.shape), _full_spec(b1.shape),
            _full_spec(conv2_weight.shape),
        ],
        out_specs=_row_spec(D_IN),
        out_shape=jax.ShapeDtypeStruct((N_PAD, D_IN), jnp.float32),
    )(d0, d1, acc1[0], acc1[1], xp, conv1_weight, b1, conv2_weight)

    acc2 = _sc_scatter(gp, row, col)

    out = pl.pallas_call(
        _final_body,
        grid=grid,
        in_specs=[
            _row_spec(DEG_W), _row_spec(DEG_W),
            _row_spec(D_IN), _row_spec(D_IN), _row_spec(D_IN),
            _full_spec(b2.shape),
        ],
        out_specs=_row_spec(D_IN),
        out_shape=jax.ShapeDtypeStruct((N_PAD, D_IN), jnp.float32),
    )(d0, d1, acc2[0], acc2[1], gp, b2)

    return out[:N_NODES]


# speedup vs baseline: 2.1274x; 1.0019x over previous
import functools

import jax
import jax.numpy as jnp
from jax import lax
from jax.experimental import pallas as pl
from jax.experimental.pallas import tpu as pltpu
from jax.experimental.pallas import tpu_sc as plsc

N_NODES = 10000
N_PAD = 10240
D_IN = 128
E_EDGES = 320000
CH = 128
NCHUNK = 80
E_PAD = 2 * 16 * NCHUNK * CH
DEG_W = 128
DEG_OUT = 128
ROWS_PER_TILE = N_PAD // 16

_MESH = plsc.VectorSubcoreMesh(core_axis_name="c", subcore_axis_name="s")


@functools.partial(
    pl.kernel,
    mesh=_MESH,
    out_type=jax.ShapeDtypeStruct((2, N_PAD, DEG_OUT), jnp.float32),
    scratch_types=[
        pltpu.VMEM((NCHUNK, CH), jnp.int32),
        pltpu.VMEM((CH, DEG_W), jnp.float32),
        pltpu.VMEM((CH, DEG_W), jnp.float32),
        pltpu.VMEM_SHARED((N_PAD, DEG_W), jnp.float32),
        pltpu.SemaphoreType.DMA,
    ],
)
def _sc_degree(col_hbm, out_hbm, cidx, ones_v, zeros_v, dacc, g0):
    c = lax.axis_index("c")
    s = lax.axis_index("s")
    one16 = jnp.ones((16,), jnp.float32)
    zero16 = jnp.zeros((16,), jnp.float32)

    def _fill(i, _):
        for j in range(DEG_W // 16):
            ones_v[i, pl.ds(j * 16, 16)] = one16
            zeros_v[i, pl.ds(j * 16, 16)] = zero16
        return 0

    lax.fori_loop(0, CH, _fill, 0)
    for blk in range(ROWS_PER_TILE // CH):
        pltpu.sync_copy(zeros_v, dacc.at[pl.ds(s * ROWS_PER_TILE + blk * CH, CH)])
    pltpu.sync_copy(col_hbm.at[c, s], cidx)
    plsc.subcore_barrier()

    def _fire(j, _):
        pltpu.async_copy(ones_v, dacc.at[cidx.at[j]], g0, add=True)
        return 0

    lax.fori_loop(0, NCHUNK, _fire, 0)

    def _drain(j, _):
        pltpu.make_async_copy(ones_v, dacc.at[cidx.at[j]], g0).wait()
        return 0

    lax.fori_loop(0, NCHUNK, _drain, 0)
    plsc.subcore_barrier()
    pltpu.sync_copy(
        dacc.at[pl.ds(s * ROWS_PER_TILE, ROWS_PER_TILE), pl.ds(0, DEG_OUT)],
        out_hbm.at[c, pl.ds(s * ROWS_PER_TILE, ROWS_PER_TILE)])


@functools.partial(
    pl.kernel,
    mesh=_MESH,
    out_type=jax.ShapeDtypeStruct((2, N_PAD, D_IN), jnp.float32),
    scratch_types=[
        pltpu.VMEM((NCHUNK // 2, CH), jnp.int32),
        pltpu.VMEM((NCHUNK // 2, CH), jnp.int32),
        pltpu.VMEM((CH, D_IN), jnp.float32),
        pltpu.VMEM((CH, D_IN), jnp.float32),
        pltpu.VMEM_SHARED((N_PAD, D_IN), jnp.float32),
        pltpu.SemaphoreType.DMA,
        pltpu.SemaphoreType.DMA,
    ],
)
def _sc_scatter(vals_hbm, row_hbm, col_hbm, out_hbm, ridx, cidx, rows0, rows1,
                acc, g0, g1):
    c = lax.axis_index("c")
    s = lax.axis_index("s")
    zero16 = jnp.zeros((16,), jnp.float32)
    half = NCHUNK // 2

    def _zrow(i, _):
        for j in range(D_IN // 16):
            rows0[i, pl.ds(j * 16, 16)] = zero16
        return 0

    lax.fori_loop(0, CH, _zrow, 0)
    for blk in range(ROWS_PER_TILE // CH):
        pltpu.sync_copy(rows0, acc.at[pl.ds(s * ROWS_PER_TILE + blk * CH, CH)])
    plsc.subcore_barrier()

    for h in range(2):
        pltpu.sync_copy(row_hbm.at[c, s, h], ridx)
        pltpu.sync_copy(col_hbm.at[c, s, h], cidx)
        pltpu.async_copy(vals_hbm.at[ridx.at[0]], rows0, g0)
        pltpu.async_copy(vals_hbm.at[ridx.at[1]], rows1, g1)

        def _body(k, _):
            j = 2 * k
            pltpu.make_async_copy(vals_hbm.at[ridx.at[j]], rows0, g0).wait()
            pltpu.sync_copy(rows0, acc.at[cidx.at[j]], add=True)
            pltpu.async_copy(vals_hbm.at[ridx.at[j + 2]], rows0, g0)
            pltpu.make_async_copy(vals_hbm.at[ridx.at[j + 1]], rows1, g1).wait()
            pltpu.sync_copy(rows1, acc.at[cidx.at[j + 1]], add=True)
            pltpu.async_copy(vals_hbm.at[ridx.at[j + 3]], rows1, g1)
            return 0

        lax.fori_loop(0, half // 2 - 1, _body, 0)
        jlast = half - 2
        pltpu.make_async_copy(vals_hbm.at[ridx.at[jlast]], rows0, g0).wait()
        pltpu.sync_copy(rows0, acc.at[cidx.at[jlast]], add=True)
        pltpu.make_async_copy(vals_hbm.at[ridx.at[jlast + 1]], rows1, g1).wait()
        pltpu.sync_copy(rows1, acc.at[cidx.at[jlast + 1]], add=True)
    plsc.subcore_barrier()
    pltpu.sync_copy(acc.at[pl.ds(s * ROWS_PER_TILE, ROWS_PER_TILE)],
                    out_hbm.at[c, pl.ds(s * ROWS_PER_TILE, ROWS_PER_TILE)])


_RB = 1280


def _dis_block(d0, d1):
    deg = d0[:, 0:1] + d1[:, 0:1] + 1.0
    return lax.rsqrt(deg)


def _prescale_body(d0, d1, x_ref, o_ref):
    o_ref[...] = x_ref[...] * _dis_block(d0, d1)


def _mid_body(d0, d1, a0, a1, xp_ref, w1, b1, w2, o_ref):
    dis = _dis_block(d0, d1)
    s1 = (a0[...] + a1[...] + xp_ref[...]) * dis
    h = jnp.maximum(
        jnp.dot(s1, w1[...], preferred_element_type=jnp.float32) + b1[...], 0.0)
    g = jnp.dot(h, w2[...], preferred_element_type=jnp.float32)
    o_ref[...] = g * dis


def _final_body(d0, d1, a0, a1, gp_ref, b2, o_ref):
    dis = _dis_block(d0, d1)
    o_ref[...] = (a0[...] + a1[...] + gp_ref[...]) * dis + b2[...]


def _row_spec(width):
    return pl.BlockSpec((_RB, width), lambda i: (i, 0))


def _full_spec(shape):
    return pl.BlockSpec(shape, lambda i: tuple(0 for _ in shape))


def kernel(x, edge_index, conv1_weight, conv1_bias, conv2_weight, conv2_bias):
    ei = edge_index.astype(jnp.int32)
    pad_i = jnp.arange(E_PAD - E_EDGES, dtype=jnp.int32)
    pad_src = pad_i % N_PAD
    pad_dst = N_NODES + pad_i % (N_PAD - N_NODES)
    row = jnp.concatenate([ei[0], pad_src]).reshape(2, 16, 2, NCHUNK // 2, CH)
    col = jnp.concatenate([ei[1], pad_dst]).reshape(2, 16, 2, NCHUNK // 2, CH)
    col4 = col.reshape(2, 16, NCHUNK, CH)
    x_pad = jnp.zeros((N_PAD, D_IN), x.dtype).at[:N_NODES].set(x)
    b1 = conv1_bias.reshape(1, -1)
    b2 = conv2_bias.reshape(1, -1)

    deg2 = _sc_degree(col4)
    d0, d1 = deg2[0], deg2[1]

    grid = (N_PAD // _RB,)
    xp = pl.pallas_call(
        _prescale_body,
        grid=grid,
        in_specs=[_row_spec(DEG_OUT), _row_spec(DEG_OUT), _row_spec(D_IN)],
        out_specs=_row_spec(D_IN),
        out_shape=jax.ShapeDtypeStruct((N_PAD, D_IN), jnp.float32),
    )(d0, d1, x_pad)

    acc1 = _sc_scatter(xp, row, col)

    gp = pl.pallas_call(
        _mid_body,
        grid=grid,
        in_specs=[
            _row_spec(DEG_OUT), _row_spec(DEG_OUT),
            _row_spec(D_IN), _row_spec(D_IN), _row_spec(D_IN),
            _full_spec(conv1_weight.shape), _full_spec(b1.shape),
            _full_spec(conv2_weight.shape),
        ],
        out_specs=_row_spec(D_IN),
        out_shape=jax.ShapeDtypeStruct((N_PAD, D_IN), jnp.float32),
    )(d0, d1, acc1[0], acc1[1], xp, conv1_weight, b1, conv2_weight)

    acc2 = _sc_scatter(gp, row, col)

    out = pl.pallas_call(
        _final_body,
        grid=grid,
        in_specs=[
            _row_spec(DEG_OUT), _row_spec(DEG_OUT),
            _row_spec(D_IN), _row_spec(D_IN), _row_spec(D_IN),
            _full_spec(b2.shape),
        ],
        out_specs=_row_spec(D_IN),
        out_shape=jax.ShapeDtypeStruct((N_PAD, D_IN), jnp.float32),
    )(d0, d1, acc2[0], acc2[1], gp, b2)

    return out[:N_NODES]


# speedup vs baseline: 2.1351x; 1.0036x over previous
import functools

import jax
import jax.numpy as jnp
from jax import lax
from jax.experimental import pallas as pl
from jax.experimental.pallas import tpu as pltpu
from jax.experimental.pallas import tpu_sc as plsc

N_NODES = 10000
N_PAD = 10240
D_IN = 128
E_EDGES = 320000
CH = 128
NCHUNK = 80
E_PAD = 2 * 16 * NCHUNK * CH
DEG_W = 128
DEG_OUT = 128
ROWS_PER_TILE = N_PAD // 16

_MESH = plsc.VectorSubcoreMesh(core_axis_name="c", subcore_axis_name="s")


@functools.partial(
    pl.kernel,
    mesh=_MESH,
    out_type=jax.ShapeDtypeStruct((2, N_PAD, DEG_OUT), jnp.float32),
    scratch_types=[
        pltpu.VMEM((NCHUNK, CH), jnp.int32),
        pltpu.VMEM((CH, DEG_W), jnp.float32),
        pltpu.VMEM((CH, DEG_W), jnp.float32),
        pltpu.VMEM_SHARED((N_PAD, DEG_W), jnp.float32),
        pltpu.SemaphoreType.DMA,
    ],
)
def _sc_degree(col_hbm, out_hbm, cidx, ones_v, zeros_v, dacc, g0):
    c = lax.axis_index("c")
    s = lax.axis_index("s")
    one16 = jnp.ones((16,), jnp.float32)
    zero16 = jnp.zeros((16,), jnp.float32)

    def _fill(i, _):
        for j in range(DEG_W // 16):
            ones_v[i, pl.ds(j * 16, 16)] = one16
            zeros_v[i, pl.ds(j * 16, 16)] = zero16
        return 0

    lax.fori_loop(0, CH, _fill, 0)
    for blk in range(ROWS_PER_TILE // CH):
        pltpu.sync_copy(zeros_v, dacc.at[pl.ds(s * ROWS_PER_TILE + blk * CH, CH)])
    pltpu.sync_copy(col_hbm.at[c, s], cidx)
    plsc.subcore_barrier()

    def _fire(j, _):
        pltpu.async_copy(ones_v, dacc.at[cidx.at[j]], g0, add=True)
        return 0

    lax.fori_loop(0, NCHUNK, _fire, 0)

    def _drain(j, _):
        pltpu.make_async_copy(ones_v, dacc.at[cidx.at[j]], g0).wait()
        return 0

    lax.fori_loop(0, NCHUNK, _drain, 0)
    plsc.subcore_barrier()
    pltpu.sync_copy(
        dacc.at[pl.ds(s * ROWS_PER_TILE, ROWS_PER_TILE), pl.ds(0, DEG_OUT)],
        out_hbm.at[c, pl.ds(s * ROWS_PER_TILE, ROWS_PER_TILE)])


@functools.partial(
    pl.kernel,
    mesh=_MESH,
    out_type=jax.ShapeDtypeStruct((2, N_PAD, D_IN), jnp.float32),
    scratch_types=[
        pltpu.VMEM((NCHUNK // 2, CH), jnp.int32),
        pltpu.VMEM((NCHUNK // 2, CH), jnp.int32),
        pltpu.VMEM((CH, D_IN), jnp.float32),
        pltpu.VMEM((CH, D_IN), jnp.float32),
        pltpu.VMEM_SHARED((N_PAD, D_IN), jnp.float32),
        pltpu.SemaphoreType.DMA,
        pltpu.SemaphoreType.DMA,
    ],
)
def _sc_scatter(vals_hbm, row_hbm, col_hbm, out_hbm, ridx, cidx, rows0, rows1,
                acc, g0, g1):
    c = lax.axis_index("c")
    s = lax.axis_index("s")
    zero16 = jnp.zeros((16,), jnp.float32)
    half = NCHUNK // 2

    def _zrow(i, _):
        for j in range(D_IN // 16):
            rows0[i, pl.ds(j * 16, 16)] = zero16
        return 0

    lax.fori_loop(0, CH, _zrow, 0)
    for blk in range(ROWS_PER_TILE // CH):
        pltpu.sync_copy(rows0, acc.at[pl.ds(s * ROWS_PER_TILE + blk * CH, CH)])
    plsc.subcore_barrier()

    for h in range(2):
        pltpu.sync_copy(row_hbm.at[c, s, h], ridx)
        pltpu.sync_copy(col_hbm.at[c, s, h], cidx)
        pltpu.async_copy(vals_hbm.at[ridx.at[0]], rows0, g0)
        pltpu.async_copy(vals_hbm.at[ridx.at[1]], rows1, g1)

        def _body(k, _):
            j = 2 * k
            pltpu.make_async_copy(vals_hbm.at[ridx.at[j]], rows0, g0).wait()
            pltpu.sync_copy(rows0, acc.at[cidx.at[j]], add=True)
            pltpu.async_copy(vals_hbm.at[ridx.at[j + 2]], rows0, g0)
            pltpu.make_async_copy(vals_hbm.at[ridx.at[j + 1]], rows1, g1).wait()
            pltpu.sync_copy(rows1, acc.at[cidx.at[j + 1]], add=True)
            pltpu.async_copy(vals_hbm.at[ridx.at[j + 3]], rows1, g1)
            return 0

        lax.fori_loop(0, half // 2 - 1, _body, 0)
        jlast = half - 2
        pltpu.make_async_copy(vals_hbm.at[ridx.at[jlast]], rows0, g0).wait()
        pltpu.sync_copy(rows0, acc.at[cidx.at[jlast]], add=True)
        pltpu.make_async_copy(vals_hbm.at[ridx.at[jlast + 1]], rows1, g1).wait()
        pltpu.sync_copy(rows1, acc.at[cidx.at[jlast + 1]], add=True)
    plsc.subcore_barrier()
    pltpu.sync_copy(acc.at[pl.ds(s * ROWS_PER_TILE, ROWS_PER_TILE)],
                    out_hbm.at[c, pl.ds(s * ROWS_PER_TILE, ROWS_PER_TILE)])


_RB = 1280


def _dis_block(d0, d1):
    deg = d0[:, 0:1] + d1[:, 0:1] + 1.0
    return lax.rsqrt(deg)


def _prescale_body(d0, d1, x_ref, o_ref, dis_ref):
    dis = _dis_block(d0, d1)
    o_ref[...] = x_ref[...] * dis
    dis_ref[...] = jnp.broadcast_to(dis, dis_ref.shape)


def _mid_body(dis_ref, a0, a1, xp_ref, w1, b1, w2, o_ref):
    dis = dis_ref[:, 0:1]
    s1 = (a0[...] + a1[...] + xp_ref[...]) * dis
    h = jnp.maximum(
        jnp.dot(s1, w1[...], preferred_element_type=jnp.float32) + b1[...], 0.0)
    g = jnp.dot(h, w2[...], preferred_element_type=jnp.float32)
    o_ref[...] = g * dis


def _final_body(dis_ref, a0, a1, gp_ref, b2, o_ref):
    dis = dis_ref[:, 0:1]
    o_ref[...] = (a0[...] + a1[...] + gp_ref[...]) * dis + b2[...]


def _row_spec(width):
    return pl.BlockSpec((_RB, width), lambda i: (i, 0))


def _full_spec(shape):
    return pl.BlockSpec(shape, lambda i: tuple(0 for _ in shape))


def kernel(x, edge_index, conv1_weight, conv1_bias, conv2_weight, conv2_bias):
    ei = edge_index.astype(jnp.int32)
    pad_i = jnp.arange(E_PAD - E_EDGES, dtype=jnp.int32)
    pad_src = pad_i % N_PAD
    pad_dst = N_NODES + pad_i % (N_PAD - N_NODES)
    row = jnp.concatenate([ei[0], pad_src]).reshape(2, 16, 2, NCHUNK // 2, CH)
    col = jnp.concatenate([ei[1], pad_dst]).reshape(2, 16, 2, NCHUNK // 2, CH)
    col4 = col.reshape(2, 16, NCHUNK, CH)
    x_pad = jnp.zeros((N_PAD, D_IN), x.dtype).at[:N_NODES].set(x)
    b1 = conv1_bias.reshape(1, -1)
    b2 = conv2_bias.reshape(1, -1)

    deg2 = _sc_degree(col4)
    d0, d1 = deg2[0], deg2[1]

    grid = (N_PAD // _RB,)
    xp, disb = pl.pallas_call(
        _prescale_body,
        grid=grid,
        in_specs=[_row_spec(DEG_OUT), _row_spec(DEG_OUT), _row_spec(D_IN)],
        out_specs=(_row_spec(D_IN), _row_spec(D_IN)),
        out_shape=(jax.ShapeDtypeStruct((N_PAD, D_IN), jnp.float32),
                   jax.ShapeDtypeStruct((N_PAD, D_IN), jnp.float32)),
    )(d0, d1, x_pad)

    acc1 = _sc_scatter(xp, row, col)

    gp = pl.pallas_call(
        _mid_body,
        grid=grid,
        in_specs=[
            _row_spec(D_IN),
            _row_spec(D_IN), _row_spec(D_IN), _row_spec(D_IN),
            _full_spec(conv1_weight.shape), _full_spec(b1.shape),
            _full_spec(conv2_weight.shape),
        ],
        out_specs=_row_spec(D_IN),
        out_shape=jax.ShapeDtypeStruct((N_PAD, D_IN), jnp.float32),
    )(disb, acc1[0], acc1[1], xp, conv1_weight, b1, conv2_weight)

    acc2 = _sc_scatter(gp, row, col)

    out = pl.pallas_call(
        _final_body,
        grid=grid,
        in_specs=[
            _row_spec(D_IN),
            _row_spec(D_IN), _row_spec(D_IN), _row_spec(D_IN),
            _full_spec(b2.shape),
        ],
        out_specs=_row_spec(D_IN),
        out_shape=jax.ShapeDtypeStruct((N_PAD, D_IN), jnp.float32),
    )(disb, acc2[0], acc2[1], gp, b2)

    return out[:N_NODES]


# speedup vs baseline: 2.1663x; 1.0146x over previous
import functools

import jax
import jax.numpy as jnp
from jax import lax
from jax.experimental import pallas as pl
from jax.experimental.pallas import tpu as pltpu
from jax.experimental.pallas import tpu_sc as plsc

N_NODES = 10000
N_PAD = 10240
D_IN = 128
E_EDGES = 320000
CH = 128
NCHUNK = 80
E_PAD = 2 * 16 * NCHUNK * CH
DEG_W = 128
DEG_OUT = 128
ROWS_PER_TILE = N_PAD // 16

_MESH = plsc.VectorSubcoreMesh(core_axis_name="c", subcore_axis_name="s")


@functools.partial(
    pl.kernel,
    mesh=_MESH,
    out_type=jax.ShapeDtypeStruct((2, N_PAD, DEG_OUT), jnp.float32),
    scratch_types=[
        pltpu.VMEM((NCHUNK, CH), jnp.int32),
        pltpu.VMEM((CH, DEG_W), jnp.float32),
        pltpu.VMEM((CH, DEG_W), jnp.float32),
        pltpu.VMEM_SHARED((N_PAD, DEG_W), jnp.float32),
        pltpu.SemaphoreType.DMA,
    ],
)
def _sc_degree(col_hbm, out_hbm, cidx, ones_v, zeros_v, dacc, g0):
    c = lax.axis_index("c")
    s = lax.axis_index("s")
    one16 = jnp.ones((16,), jnp.float32)
    zero16 = jnp.zeros((16,), jnp.float32)

    def _fill(i, _):
        for j in range(DEG_W // 16):
            ones_v[i, pl.ds(j * 16, 16)] = one16
            zeros_v[i, pl.ds(j * 16, 16)] = zero16
        return 0

    lax.fori_loop(0, CH, _fill, 0)
    for blk in range(ROWS_PER_TILE // CH):
        pltpu.sync_copy(zeros_v, dacc.at[pl.ds(s * ROWS_PER_TILE + blk * CH, CH)])
    pltpu.sync_copy(col_hbm.at[c, s], cidx)
    plsc.subcore_barrier()

    def _fire(j, _):
        pltpu.async_copy(ones_v, dacc.at[cidx.at[j]], g0, add=True)
        return 0

    lax.fori_loop(0, NCHUNK, _fire, 0)

    def _drain(j, _):
        pltpu.make_async_copy(ones_v, dacc.at[cidx.at[j]], g0).wait()
        return 0

    lax.fori_loop(0, NCHUNK, _drain, 0)
    plsc.subcore_barrier()
    pltpu.sync_copy(
        dacc.at[pl.ds(s * ROWS_PER_TILE, ROWS_PER_TILE), pl.ds(0, DEG_OUT)],
        out_hbm.at[c, pl.ds(s * ROWS_PER_TILE, ROWS_PER_TILE)])


@functools.partial(
    pl.kernel,
    mesh=_MESH,
    out_type=jax.ShapeDtypeStruct((2, N_PAD, D_IN), jnp.float32),
    scratch_types=[
        pltpu.VMEM((NCHUNK // 2, CH), jnp.int32),
        pltpu.VMEM((NCHUNK // 2, CH), jnp.int32),
        pltpu.VMEM((CH, D_IN), jnp.float32),
        pltpu.VMEM((CH, D_IN), jnp.float32),
        pltpu.VMEM_SHARED((N_PAD, D_IN), jnp.float32),
        pltpu.SemaphoreType.DMA,
        pltpu.SemaphoreType.DMA,
    ],
)
def _sc_scatter(vals_hbm, row_hbm, col_hbm, out_hbm, ridx, cidx, rows0, rows1,
                acc, g0, g1):
    c = lax.axis_index("c")
    s = lax.axis_index("s")
    zero16 = jnp.zeros((16,), jnp.float32)
    half = NCHUNK // 2

    def _zrow(i, _):
        for j in range(D_IN // 16):
            rows0[i, pl.ds(j * 16, 16)] = zero16
        return 0

    lax.fori_loop(0, CH, _zrow, 0)
    for blk in range(ROWS_PER_TILE // CH):
        pltpu.sync_copy(rows0, acc.at[pl.ds(s * ROWS_PER_TILE + blk * CH, CH)])
    plsc.subcore_barrier()

    for h in range(2):
        pltpu.sync_copy(row_hbm.at[c, s, h], ridx)
        pltpu.sync_copy(col_hbm.at[c, s, h], cidx)
        pltpu.async_copy(vals_hbm.at[ridx.at[0]], rows0, g0)
        pltpu.async_copy(vals_hbm.at[ridx.at[1]], rows1, g1)

        def _body(k, _):
            j = 2 * k
            pltpu.make_async_copy(vals_hbm.at[ridx.at[j]], rows0, g0).wait()
            pltpu.sync_copy(rows0, acc.at[cidx.at[j]], add=True)
            pltpu.async_copy(vals_hbm.at[ridx.at[j + 2]], rows0, g0)
            pltpu.make_async_copy(vals_hbm.at[ridx.at[j + 1]], rows1, g1).wait()
            pltpu.sync_copy(rows1, acc.at[cidx.at[j + 1]], add=True)
            pltpu.async_copy(vals_hbm.at[ridx.at[j + 3]], rows1, g1)
            return 0

        lax.fori_loop(0, half // 2 - 1, _body, 0)
        jlast = half - 2
        pltpu.make_async_copy(vals_hbm.at[ridx.at[jlast]], rows0, g0).wait()
        pltpu.sync_copy(rows0, acc.at[cidx.at[jlast]], add=True)
        pltpu.make_async_copy(vals_hbm.at[ridx.at[jlast + 1]], rows1, g1).wait()
        pltpu.sync_copy(rows1, acc.at[cidx.at[jlast + 1]], add=True)
    plsc.subcore_barrier()
    pltpu.sync_copy(acc.at[pl.ds(s * ROWS_PER_TILE, ROWS_PER_TILE)],
                    out_hbm.at[c, pl.ds(s * ROWS_PER_TILE, ROWS_PER_TILE)])


_RB = 2560


def _dis_block(d0, d1):
    deg = d0[:, 0:1] + d1[:, 0:1] + 1.0
    return lax.rsqrt(deg)


def _prescale_body(d0, d1, x_ref, o_ref, dis_ref):
    dis = _dis_block(d0, d1)
    o_ref[...] = x_ref[...] * dis
    dis_ref[...] = jnp.broadcast_to(dis, dis_ref.shape)


def _mid_body(dis_ref, a0, a1, xp_ref, w1, b1, w2, o_ref):
    dis = dis_ref[:, 0:1]
    s1 = (a0[...] + a1[...] + xp_ref[...]) * dis
    h = jnp.maximum(
        jnp.dot(s1, w1[...], preferred_element_type=jnp.float32) + b1[...], 0.0)
    g = jnp.dot(h, w2[...], preferred_element_type=jnp.float32)
    o_ref[...] = g * dis


def _final_body(dis_ref, a0, a1, gp_ref, b2, o_ref):
    dis = dis_ref[:, 0:1]
    o_ref[...] = (a0[...] + a1[...] + gp_ref[...]) * dis + b2[...]


def _row_spec(width):
    return pl.BlockSpec((_RB, width), lambda i: (i, 0))


def _full_spec(shape):
    return pl.BlockSpec(shape, lambda i: tuple(0 for _ in shape))


def kernel(x, edge_index, conv1_weight, conv1_bias, conv2_weight, conv2_bias):
    ei = edge_index.astype(jnp.int32)
    pad_i = jnp.arange(E_PAD - E_EDGES, dtype=jnp.int32)
    pad_src = pad_i % N_PAD
    pad_dst = N_NODES + pad_i % (N_PAD - N_NODES)
    row = jnp.concatenate([ei[0], pad_src]).reshape(2, 16, 2, NCHUNK // 2, CH)
    col = jnp.concatenate([ei[1], pad_dst]).reshape(2, 16, 2, NCHUNK // 2, CH)
    col4 = col.reshape(2, 16, NCHUNK, CH)
    x_pad = jnp.zeros((N_PAD, D_IN), x.dtype).at[:N_NODES].set(x)
    b1 = conv1_bias.reshape(1, -1)
    b2 = conv2_bias.reshape(1, -1)

    deg2 = _sc_degree(col4)
    d0, d1 = deg2[0], deg2[1]

    grid = (N_PAD // _RB,)
    xp, disb = pl.pallas_call(
        _prescale_body,
        grid=grid,
        in_specs=[_row_spec(DEG_OUT), _row_spec(DEG_OUT), _row_spec(D_IN)],
        out_specs=(_row_spec(D_IN), _row_spec(D_IN)),
        out_shape=(jax.ShapeDtypeStruct((N_PAD, D_IN), jnp.float32),
                   jax.ShapeDtypeStruct((N_PAD, D_IN), jnp.float32)),
    )(d0, d1, x_pad)

    acc1 = _sc_scatter(xp, row, col)

    gp = pl.pallas_call(
        _mid_body,
        grid=grid,
        in_specs=[
            _row_spec(D_IN),
            _row_spec(D_IN), _row_spec(D_IN), _row_spec(D_IN),
            _full_spec(conv1_weight.shape), _full_spec(b1.shape),
            _full_spec(conv2_weight.shape),
        ],
        out_specs=_row_spec(D_IN),
        out_shape=jax.ShapeDtypeStruct((N_PAD, D_IN), jnp.float32),
    )(disb, acc1[0], acc1[1], xp, conv1_weight, b1, conv2_weight)

    acc2 = _sc_scatter(gp, row, col)

    out = pl.pallas_call(
        _final_body,
        grid=grid,
        in_specs=[
            _row_spec(D_IN),
            _row_spec(D_IN), _row_spec(D_IN), _row_spec(D_IN),
            _full_spec(b2.shape),
        ],
        out_specs=_row_spec(D_IN),
        out_shape=jax.ShapeDtypeStruct((N_PAD, D_IN), jnp.float32),
    )(disb, acc2[0], acc2[1], gp, b2)

    return out[:N_NODES]


# speedup vs baseline: 2.1715x; 1.0024x over previous
import functools

import jax
import jax.numpy as jnp
from jax import lax
from jax.experimental import pallas as pl
from jax.experimental.pallas import tpu as pltpu
from jax.experimental.pallas import tpu_sc as plsc

N_NODES = 10000
N_PAD = 10240
D_IN = 128
E_EDGES = 320000
CH = 128
NCHUNK = 80
E_PAD = 2 * 16 * NCHUNK * CH
DEG_W = 128
DEG_OUT = 128
ROWS_PER_TILE = N_PAD // 16

_MESH = plsc.VectorSubcoreMesh(core_axis_name="c", subcore_axis_name="s")


@functools.partial(
    pl.kernel,
    mesh=_MESH,
    out_type=jax.ShapeDtypeStruct((2, N_PAD, DEG_OUT), jnp.float32),
    scratch_types=[
        pltpu.VMEM((NCHUNK, CH), jnp.int32),
        pltpu.VMEM((CH, DEG_W), jnp.float32),
        pltpu.VMEM((CH, DEG_W), jnp.float32),
        pltpu.VMEM_SHARED((N_PAD, DEG_W), jnp.float32),
        pltpu.SemaphoreType.DMA,
    ],
)
def _sc_degree(col_hbm, out_hbm, cidx, ones_v, zeros_v, dacc, g0):
    c = lax.axis_index("c")
    s = lax.axis_index("s")
    one16 = jnp.ones((16,), jnp.float32)
    zero16 = jnp.zeros((16,), jnp.float32)

    def _fill(i, _):
        for j in range(DEG_W // 16):
            ones_v[i, pl.ds(j * 16, 16)] = one16
            zeros_v[i, pl.ds(j * 16, 16)] = zero16
        return 0

    lax.fori_loop(0, CH, _fill, 0)
    for blk in range(ROWS_PER_TILE // CH):
        pltpu.sync_copy(zeros_v, dacc.at[pl.ds(s * ROWS_PER_TILE + blk * CH, CH)])
    pltpu.sync_copy(col_hbm.at[c, s], cidx)
    plsc.subcore_barrier()

    def _fire(j, _):
        pltpu.async_copy(ones_v, dacc.at[cidx.at[j]], g0, add=True)
        return 0

    lax.fori_loop(0, NCHUNK, _fire, 0)

    def _drain(j, _):
        pltpu.make_async_copy(ones_v, dacc.at[cidx.at[j]], g0).wait()
        return 0

    lax.fori_loop(0, NCHUNK, _drain, 0)
    plsc.subcore_barrier()
    pltpu.sync_copy(
        dacc.at[pl.ds(s * ROWS_PER_TILE, ROWS_PER_TILE), pl.ds(0, DEG_OUT)],
        out_hbm.at[c, pl.ds(s * ROWS_PER_TILE, ROWS_PER_TILE)])


@functools.partial(
    pl.kernel,
    mesh=_MESH,
    out_type=jax.ShapeDtypeStruct((2, N_PAD, D_IN), jnp.float32),
    scratch_types=[
        pltpu.VMEM((NCHUNK // 2, CH), jnp.int32),
        pltpu.VMEM((NCHUNK // 2, CH), jnp.int32),
        pltpu.VMEM((CH, D_IN), jnp.float32),
        pltpu.VMEM((CH, D_IN), jnp.float32),
        pltpu.VMEM_SHARED((N_PAD, D_IN), jnp.float32),
        pltpu.SemaphoreType.DMA,
        pltpu.SemaphoreType.DMA,
    ],
)
def _sc_scatter(vals_hbm, row_hbm, col_hbm, out_hbm, ridx, cidx, rows0, rows1,
                acc, g0, g1):
    c = lax.axis_index("c")
    s = lax.axis_index("s")
    zero16 = jnp.zeros((16,), jnp.float32)
    half = NCHUNK // 2

    def _zrow(i, _):
        for j in range(D_IN // 16):
            rows0[i, pl.ds(j * 16, 16)] = zero16
        return 0

    lax.fori_loop(0, CH, _zrow, 0)
    for blk in range(ROWS_PER_TILE // CH):
        pltpu.sync_copy(rows0, acc.at[pl.ds(s * ROWS_PER_TILE + blk * CH, CH)])
    plsc.subcore_barrier()

    for h in range(2):
        pltpu.sync_copy(row_hbm.at[c, s, h], ridx)
        pltpu.sync_copy(col_hbm.at[c, s, h], cidx)
        pltpu.async_copy(vals_hbm.at[ridx.at[0]], rows0, g0)
        pltpu.async_copy(vals_hbm.at[ridx.at[1]], rows1, g1)

        def _body(k, _):
            j = 2 * k
            pltpu.make_async_copy(vals_hbm.at[ridx.at[j]], rows0, g0).wait()
            pltpu.sync_copy(rows0, acc.at[cidx.at[j]], add=True)
            pltpu.async_copy(vals_hbm.at[ridx.at[j + 2]], rows0, g0)
            pltpu.make_async_copy(vals_hbm.at[ridx.at[j + 1]], rows1, g1).wait()
            pltpu.sync_copy(rows1, acc.at[cidx.at[j + 1]], add=True)
            pltpu.async_copy(vals_hbm.at[ridx.at[j + 3]], rows1, g1)
            return 0

        lax.fori_loop(0, half // 2 - 1, _body, 0)
        jlast = half - 2
        pltpu.make_async_copy(vals_hbm.at[ridx.at[jlast]], rows0, g0).wait()
        pltpu.sync_copy(rows0, acc.at[cidx.at[jlast]], add=True)
        pltpu.make_async_copy(vals_hbm.at[ridx.at[jlast + 1]], rows1, g1).wait()
        pltpu.sync_copy(rows1, acc.at[cidx.at[jlast + 1]], add=True)
    plsc.subcore_barrier()
    pltpu.sync_copy(acc.at[pl.ds(s * ROWS_PER_TILE, ROWS_PER_TILE)],
                    out_hbm.at[c, pl.ds(s * ROWS_PER_TILE, ROWS_PER_TILE)])


_RB = 5120


def _dis_block(d0, d1):
    deg = d0[:, 0:1] + d1[:, 0:1] + 1.0
    return lax.rsqrt(deg)


def _prescale_body(d0, d1, x_ref, o_ref, dis_ref):
    dis = _dis_block(d0, d1)
    o_ref[...] = x_ref[...] * dis
    dis_ref[...] = jnp.broadcast_to(dis, dis_ref.shape)


def _mid_body(dis_ref, a0, a1, xp_ref, w1, b1, w2, o_ref):
    dis = dis_ref[:, 0:1]
    s1 = (a0[...] + a1[...] + xp_ref[...]) * dis
    h = jnp.maximum(
        jnp.dot(s1, w1[...], preferred_element_type=jnp.float32) + b1[...], 0.0)
    g = jnp.dot(h, w2[...], preferred_element_type=jnp.float32)
    o_ref[...] = g * dis


def _final_body(dis_ref, a0, a1, gp_ref, b2, o_ref):
    dis = dis_ref[:, 0:1]
    o_ref[...] = (a0[...] + a1[...] + gp_ref[...]) * dis + b2[...]


def _row_spec(width):
    return pl.BlockSpec((_RB, width), lambda i: (i, 0))


def _full_spec(shape):
    return pl.BlockSpec(shape, lambda i: tuple(0 for _ in shape))


def kernel(x, edge_index, conv1_weight, conv1_bias, conv2_weight, conv2_bias):
    ei = edge_index.astype(jnp.int32)
    pad_i = jnp.arange(E_PAD - E_EDGES, dtype=jnp.int32)
    pad_src = pad_i % N_PAD
    pad_dst = N_NODES + pad_i % (N_PAD - N_NODES)
    row = jnp.concatenate([ei[0], pad_src]).reshape(2, 16, 2, NCHUNK // 2, CH)
    col = jnp.concatenate([ei[1], pad_dst]).reshape(2, 16, 2, NCHUNK // 2, CH)
    col4 = col.reshape(2, 16, NCHUNK, CH)
    x_pad = jnp.zeros((N_PAD, D_IN), x.dtype).at[:N_NODES].set(x)
    b1 = conv1_bias.reshape(1, -1)
    b2 = conv2_bias.reshape(1, -1)

    deg2 = _sc_degree(col4)
    d0, d1 = deg2[0], deg2[1]

    grid = (N_PAD // _RB,)
    xp, disb = pl.pallas_call(
        _prescale_body,
        grid=grid,
        in_specs=[_row_spec(DEG_OUT), _row_spec(DEG_OUT), _row_spec(D_IN)],
        out_specs=(_row_spec(D_IN), _row_spec(D_IN)),
        out_shape=(jax.ShapeDtypeStruct((N_PAD, D_IN), jnp.float32),
                   jax.ShapeDtypeStruct((N_PAD, D_IN), jnp.float32)),
    )(d0, d1, x_pad)

    acc1 = _sc_scatter(xp, row, col)

    gp = pl.pallas_call(
        _mid_body,
        grid=grid,
        in_specs=[
            _row_spec(D_IN),
            _row_spec(D_IN), _row_spec(D_IN), _row_spec(D_IN),
            _full_spec(conv1_weight.shape), _full_spec(b1.shape),
            _full_spec(conv2_weight.shape),
        ],
        out_specs=_row_spec(D_IN),
        out_shape=jax.ShapeDtypeStruct((N_PAD, D_IN), jnp.float32),
    )(disb, acc1[0], acc1[1], xp, conv1_weight, b1, conv2_weight)

    acc2 = _sc_scatter(gp, row, col)

    out = pl.pallas_call(
        _final_body,
        grid=grid,
        in_specs=[
            _row_spec(D_IN),
            _row_spec(D_IN), _row_spec(D_IN), _row_spec(D_IN),
            _full_spec(b2.shape),
        ],
        out_specs=_row_spec(D_IN),
        out_shape=jax.ShapeDtypeStruct((N_PAD, D_IN), jnp.float32),
    )(disb, acc2[0], acc2[1], gp, b2)

    return out[:N_NODES]
